# Initial kernel scaffold; baseline (speedup 1.0000x reference)
import jax
import jax.numpy as jnp
from jax.experimental import pallas as pl


def kernel(x, edge_index, batch, target, params):
    raise NotImplementedError("write your pallas kernel here")


# speedup vs baseline: 4.7446x; 4.7446x over previous
import functools

import jax
import jax.numpy as jnp
from jax import lax
from jax.experimental import pallas as pl
from jax.experimental.pallas import tpu as pltpu
from jax.experimental.pallas import tpu_sc as plsc

N = 50000
E = 800000
B = 128
DIM = 32
BLK = 2000
NBLK = N // BLK
F80 = 80

_NTILES = 32
_CPT = 200
_EPT = _CPT * 128
_EPAD = _NTILES * _EPT
_GRP = 4
_NGRP = _CPT // _GRP
_NACC = 50048


def _f32(x):
    return x.astype(jnp.float32)


@functools.lru_cache(maxsize=None)
def _agg_kernel(width, nrows):
    mesh = plsc.VectorSubcoreMesh(core_axis_name="c", subcore_axis_name="s")

    @functools.partial(
        pl.kernel,
        mesh=mesh,
        out_type=jax.ShapeDtypeStruct((2, _NACC, width), jnp.float32),
        compiler_params=pltpu.CompilerParams(use_tc_tiling_on_sc=False),
        scratch_types=[
            pltpu.VMEM((_GRP, 128), jnp.int32),
            pltpu.VMEM((_GRP, 128), jnp.int32),
            pltpu.VMEM((_GRP * 128, width), jnp.float32),
            pltpu.VMEM_SHARED((_NACC, width), jnp.float32),
            pltpu.SemaphoreType.DMA,
        ],
    )
    def agg(y_hbm, src_hbm, dst_hbm, z_hbm, out_hbm, idx_s, idx_d, rows, acc, sem):
        cid = lax.axis_index("c")
        sid = lax.axis_index("s")
        wid = cid * 16 + sid
        zr = _NACC // 16
        pltpu.sync_copy(z_hbm.at[pl.ds(sid * zr, zr)], acc.at[pl.ds(sid * zr, zr)])
        plsc.subcore_barrier()
        chunk0 = wid * _CPT

        def body(g, carry):
            row0 = chunk0 + g * _GRP
            pltpu.sync_copy(src_hbm.at[pl.ds(row0, _GRP)], idx_s)
            pltpu.sync_copy(dst_hbm.at[pl.ds(row0, _GRP)], idx_d)
            handles = [
                pltpu.async_copy(y_hbm.at[idx_s.at[j]],
                                 rows.at[pl.ds(j * 128, 128)], sem)
                for j in range(_GRP)
            ]
            for h in handles:
                h.wait()
            for j in range(_GRP):
                pltpu.sync_copy(rows.at[pl.ds(j * 128, 128)],
                                acc.at[idx_d.at[j]], add=True)
            return carry

        lax.fori_loop(0, _NGRP, body, 0)
        plsc.subcore_barrier()
        pltpu.sync_copy(acc.at[pl.ds(sid * zr, zr)],
                        out_hbm.at[cid].at[pl.ds(sid * zr, zr)])

    return agg


def _agg(y, srcp, dstp, zacc):
    return _agg_kernel(y.shape[1], y.shape[0])(y, srcp, dstp, zacc)


def _bn_body(u_ref, mu_ref, var_ref, g_ref, be_ref, h_ref):
    den = jnp.sqrt(var_ref[...] + 1e-5)
    h_ref[...] = (u_ref[...] - mu_ref[...]) / den * g_ref[...] + be_ref[...]


def _bn(u, mu, var, g, be):
    return pl.pallas_call(
        _bn_body,
        grid=(NBLK,),
        in_specs=[pl.BlockSpec((BLK, DIM), lambda i: (i, 0)),
                  pl.BlockSpec((1, DIM), lambda i: (0, 0)),
                  pl.BlockSpec((1, DIM), lambda i: (0, 0)),
                  pl.BlockSpec((1, DIM), lambda i: (0, 0)),
                  pl.BlockSpec((1, DIM), lambda i: (0, 0))],
        out_specs=pl.BlockSpec((BLK, DIM), lambda i: (i, 0)),
        out_shape=jax.ShapeDtypeStruct((N, DIM), jnp.float32),
    )(u, mu, var, g, be)


def _stats_update(st_ref, st):
    @pl.when(pl.program_id(0) == 0)
    def _init():
        st_ref[...] = st

    @pl.when(pl.program_id(0) > 0)
    def _acc():
        st_ref[...] += st


def _mid_body(h_ref, ag_ref, ba_ref, wa_ref, wb_ref, bb_ref, u_ref, st_ref):
    a = ag_ref[...]
    z = h_ref[...] + a[0] + a[1]
    m = jnp.maximum(
        jnp.dot(z, wa_ref[...], preferred_element_type=jnp.float32)
        + ba_ref[...], 0.0)
    u = jnp.maximum(
        jnp.dot(m, wb_ref[...], preferred_element_type=jnp.float32)
        + bb_ref[...], 0.0)
    u_ref[...] = u
    _stats_update(st_ref, jnp.stack([jnp.sum(u, axis=0),
                                     jnp.sum(u * u, axis=0)]))


def _mid(h, ag, ba, Wa, Wb, bb):
    return pl.pallas_call(
        _mid_body,
        grid=(NBLK,),
        in_specs=[pl.BlockSpec((BLK, DIM), lambda i: (i, 0)),
                  pl.BlockSpec((2, BLK, DIM), lambda i: (0, i, 0)),
                  pl.BlockSpec((1, DIM), lambda i: (0, 0)),
                  pl.BlockSpec((DIM, DIM), lambda i: (0, 0)),
                  pl.BlockSpec((DIM, DIM), lambda i: (0, 0)),
                  pl.BlockSpec((1, DIM), lambda i: (0, 0))],
        out_specs=[pl.BlockSpec((BLK, DIM), lambda i: (i, 0)),
                   pl.BlockSpec((2, DIM), lambda i: (0, 0))],
        out_shape=[jax.ShapeDtypeStruct((N, DIM), jnp.float32),
                   jax.ShapeDtypeStruct((2, DIM), jnp.float32)],
    )(h, ag, ba, Wa, Wb, bb)


def _mid1_body(x_ref, a0_ref, a1_ref, a2_ref, a3_ref, a4_ref,
               ba_ref, wa_ref, wb_ref, bb_ref, u_ref, st_ref):
    parts = [r[...] for r in (a0_ref, a1_ref, a2_ref, a3_ref, a4_ref)]
    agg = jnp.concatenate([p[0] + p[1] for p in parts], axis=1)
    z = x_ref[...] + agg
    m = jnp.maximum(
        jnp.dot(z, wa_ref[...], preferred_element_type=jnp.float32)
        + ba_ref[...], 0.0)
    u = jnp.maximum(
        jnp.dot(m, wb_ref[...], preferred_element_type=jnp.float32)
        + bb_ref[...], 0.0)
    u_ref[...] = u
    _stats_update(st_ref, jnp.stack([jnp.sum(u, axis=0),
                                     jnp.sum(u * u, axis=0)]))


def _mid1(x80, ags, ba, Wa80, Wb, bb):
    aspec = pl.BlockSpec((2, BLK, 16), lambda i: (0, i, 0))
    return pl.pallas_call(
        _mid1_body,
        grid=(NBLK,),
        in_specs=[pl.BlockSpec((BLK, F80), lambda i: (i, 0)),
                  aspec, aspec, aspec, aspec, aspec,
                  pl.BlockSpec((1, DIM), lambda i: (0, 0)),
                  pl.BlockSpec((F80, DIM), lambda i: (0, 0)),
                  pl.BlockSpec((DIM, DIM), lambda i: (0, 0)),
                  pl.BlockSpec((1, DIM), lambda i: (0, 0))],
        out_specs=[pl.BlockSpec((BLK, DIM), lambda i: (i, 0)),
                   pl.BlockSpec((2, DIM), lambda i: (0, 0))],
        out_shape=[jax.ShapeDtypeStruct((N, DIM), jnp.float32),
                   jax.ShapeDtypeStruct((2, DIM), jnp.float32)],
    )(x80, *ags, ba, Wa80, Wb, bb)


def _pool_body(u_ref, b_ref, pooled_ref, cnt_ref):
    bcol = b_ref[...][0, 0][:, None]
    iot = lax.broadcasted_iota(jnp.int32, (BLK, B), 1)
    oh = _f32(bcol == iot)
    pp = lax.dot_general(oh, u_ref[...], (((0,), (0,)), ((), ())),
                         preferred_element_type=jnp.float32,
                         precision=lax.Precision.HIGHEST)
    cc = jnp.sum(oh, axis=0)[:, None]

    @pl.when(pl.program_id(0) == 0)
    def _init():
        pooled_ref[...] = pp
        cnt_ref[...] = cc

    @pl.when(pl.program_id(0) > 0)
    def _acc():
        pooled_ref[...] += pp
        cnt_ref[...] += cc


def _pool(u, batch3):
    return pl.pallas_call(
        _pool_body,
        grid=(NBLK,),
        in_specs=[pl.BlockSpec((BLK, DIM), lambda i: (i, 0)),
                  pl.BlockSpec((1, 1, BLK), lambda i: (i, 0, 0))],
        out_specs=[pl.BlockSpec((B, DIM), lambda i: (0, 0)),
                   pl.BlockSpec((B, 1), lambda i: (0, 0))],
        out_shape=[jax.ShapeDtypeStruct((B, DIM), jnp.float32),
                   jax.ShapeDtypeStruct((B, 1), jnp.float32)],
    )(u, batch3)


def _prot_s_body(tg_ref, wcp_ref, s_ref):
    v = pl.program_id(0)
    mask = _f32(tg_ref[...] == v)
    s_ref[...] = lax.dot_general(
        wcp_ref[...], mask, (((0,), (1,)), ((), ())),
        preferred_element_type=jnp.float32)[None]


def _prot_s(target, Wcp):
    return pl.pallas_call(
        _prot_s_body,
        grid=(26,),
        in_specs=[pl.BlockSpec((B, 1000), lambda v: (0, 0)),
                  pl.BlockSpec((1000, 256), lambda v: (0, 0))],
        out_specs=pl.BlockSpec((1, 256, B), lambda v: (v, 0, 0)),
        out_shape=jax.ShapeDtypeStruct((26, 256, B), jnp.float32),
    )(target, Wcp)


def _conv_body(s_ref, ew_ref, wxt_ref, bc_ref, bxt_ref, xt_ref):
    a = s_ref[...].reshape(26 * 8, B)
    conv_o = lax.dot_general(a, ew_ref[...], (((0,), (0,)), ((), ())),
                             preferred_element_type=jnp.float32,
                             precision=lax.Precision.HIGHEST)
    conv_o = conv_o + bc_ref[...][0]
    part = jnp.dot(conv_o, wxt_ref[...][0],
                   preferred_element_type=jnp.float32)

    @pl.when(pl.program_id(0) == 0)
    def _init():
        xt_ref[...] = part + bxt_ref[...]

    @pl.when(pl.program_id(0) > 0)
    def _acc():
        xt_ref[...] += part


def _prot_conv(S, EwinF, Wxt3, bc, bxt):
    return pl.pallas_call(
        _conv_body,
        grid=(32,),
        in_specs=[pl.BlockSpec((26, 8, B), lambda o: (0, o, 0)),
                  pl.BlockSpec((26 * 8, 121), lambda o: (0, 0)),
                  pl.BlockSpec((1, 121, B), lambda o: (o, 0, 0)),
                  pl.BlockSpec((1, 1, 121), lambda o: (o, 0, 0)),
                  pl.BlockSpec((1, B), lambda o: (0, 0))],
        out_specs=pl.BlockSpec((B, B), lambda o: (0, 0)),
        out_shape=jax.ShapeDtypeStruct((B, B), jnp.float32),
    )(S, EwinF, Wxt3, bc, bxt)


def _head_body(raw_ref, cnt_ref, a_ref, c_ref, wxd_ref, bxd_ref, xt_ref,
               wf1a_ref, wf1b_ref, bf1_ref, wf2_ref, bf2_ref,
               wo_ref, bo_ref, o_ref):
    pooled = raw_ref[...] * a_ref[...] + cnt_ref[...] * c_ref[...]
    xd = jnp.maximum(
        jnp.dot(pooled, wxd_ref[...], preferred_element_type=jnp.float32)
        + bxd_ref[...], 0.0)
    h1 = jnp.maximum(
        jnp.dot(xd, wf1a_ref[...], preferred_element_type=jnp.float32)
        + jnp.dot(xt_ref[...], wf1b_ref[...], preferred_element_type=jnp.float32)
        + bf1_ref[...], 0.0)
    h2 = jnp.maximum(
        jnp.dot(h1, wf2_ref[...], preferred_element_type=jnp.float32)
        + bf2_ref[...], 0.0)
    o_ref[...] = jnp.dot(h2, wo_ref[...],
                         preferred_element_type=jnp.float32) + bo_ref[...]


def _head(raw, cnt, A, C, Wxd, bxd, xt, Wf1a, Wf1b, bf1, Wf2, bf2, Wo, bo):
    full = lambda *shape: pl.BlockSpec(shape, lambda: tuple(0 for _ in shape))
    args = (raw, cnt, A, C, Wxd, bxd, xt, Wf1a, Wf1b, bf1, Wf2, bf2, Wo, bo)
    return pl.pallas_call(
        _head_body,
        in_specs=[full(*a.shape) for a in args],
        out_specs=full(B, 1),
        out_shape=jax.ShapeDtypeStruct((B, 1), jnp.float32),
    )(*args)


def kernel(x, edge_index, batch, target, params):
    p = params
    src, dst = edge_index[0], edge_index[1]
    pad = _EPAD - E
    srcp = jnp.concatenate([src, jnp.zeros((pad,), jnp.int32)]).reshape(-1, 128)
    dstp = jnp.concatenate([dst, jnp.full((pad,), N, jnp.int32)]).reshape(-1, 128)
    zacc32 = jnp.zeros((_NACC, DIM), jnp.float32)
    zacc16 = jnp.zeros((_NACC, 16), jnp.float32)

    x80 = jnp.pad(x, ((0, 0), (0, F80 - x.shape[1])))
    x5 = x80.reshape(N * 5, 16)
    ags = [_agg(x5, srcp * 5 + j, dstp, zacc16) for j in range(5)]
    Wa80 = jnp.pad(p['W1a'], ((0, F80 - x.shape[1]), (0, 0)))
    u, st = _mid1(x80, ags, p['b1a'][None, :], Wa80, p['W1b'],
                  p['b1b'][None, :])

    for l in range(2, 7):
        mu = st[0] / N
        var = st[1] / N - mu * mu
        g, be = p['g%d' % (l - 1)], p['be%d' % (l - 1)]
        if l == 6:
            break
        h = _bn(u, mu[None, :], var[None, :], g[None, :], be[None, :])
        ag = _agg(h, srcp, dstp, zacc32)
        u, st = _mid(h, ag, p['b%da' % l][None, :], p['W%da' % l],
                     p['W%db' % l], p['b%db' % l][None, :])

    A = g / jnp.sqrt(var + 1e-5)
    C = be - mu * A
    raw, cnt = _pool(u, batch.reshape(NBLK, 1, BLK))

    Wcp = p['Wc'].transpose(1, 0, 2).reshape(1000, 256)
    Ewin = jnp.stack([p['Etab'][:, k:k + 121] for k in range(8)], axis=1)
    EwinF = Ewin.astype(jnp.bfloat16).astype(jnp.float32).reshape(26 * 8, 121)
    S = _prot_s(target, Wcp)
    bc3 = jnp.broadcast_to(p['bc'][:, None, None], (32, 1, 121))
    xt = _prot_conv(S, EwinF, p['Wxt'].reshape(32, 121, B),
                    bc3, p['bxt'][None, :])

    return _head(raw, cnt, A[None, :], C[None, :], p['Wxd'],
                 p['bxd'][None, :], xt,
                 p['Wf1'][:128], p['Wf1'][128:], p['bf1'][None, :],
                 p['Wf2'], p['bf2'][None, :], p['Wo'], p['bo'][None, :])
